# Initial kernel scaffold; baseline (speedup 1.0000x reference)
import jax
import jax.numpy as jnp
from jax.experimental import pallas as pl


def kernel(x, transformation):
    raise NotImplementedError("write your pallas kernel here")


# speedup vs baseline: 1.3150x; 1.3150x over previous
import jax
import jax.numpy as jnp
from jax import lax
from jax.experimental import pallas as pl
from jax.experimental.pallas import tpu as pltpu
from jax.experimental.pallas import tpu_sc as plsc

H = 384
W = 384
C = 96
B = 16
NPIX = B * H * W
NC = 2
NS = 16
L = 16
NW = NC * NS
ROWS_PER_W = (B * H) // NW
K = 128
GROUPS = K // L
CHUNKS_PER_ROW = W // K

_DN = lax.GatherDimensionNumbers(
    offset_dims=(), collapsed_slice_dims=(0,), start_index_map=(0,))


def _splat(vec, i):
    idx = jnp.full((L, 1), i, dtype=jnp.int32)
    return lax.gather(vec, idx, _DN, slice_sizes=(1,),
                      mode=lax.GatherScatterMode.PROMISE_IN_BOUNDS)


def _rbf(v):
    u = lax.bitcast_convert_type(v, jnp.int32)
    r = (u + jnp.int32(0x7FFF) + ((u >> 16) & 1)) & jnp.int32(-65536)
    return lax.bitcast_convert_type(r, jnp.float32)


def _body(x_hbm, t_hbm, grid_hbm, out_hbm,
          grid_v, t_v,
          ia_v, ib_v, ic_v, id_v,
          wa_v, wb_v, wc_v, wd_v,
          va_v, vb_v, vc_v, vd_v, o_v, sem):
    cid = lax.axis_index("c")
    sid = lax.axis_index("s")
    wid = sid * NC + cid
    b = wid // (NW // B)
    base = b * (H * W)

    pltpu.sync_copy(grid_hbm, grid_v)
    pltpu.sync_copy(t_hbm, t_v)

    for gg in range(W // L):
        slg = pl.ds(gg * L, L)
        grid_v[slg] = _rbf(grid_v[slg])

    T = [_rbf(_splat(t_v[pl.ds(r * L, L)], b)) for r in range(6)]

    @pl.loop(0, ROWS_PER_W)
    def _row(r):
        gi = wid * ROWS_PER_W + r
        iy = gi % H
        a16 = (iy // L) * L
        ygs = _splat(grid_v[pl.ds(a16, L)], iy % L)
        xb = T[1] * ygs + T[2]
        yb = T[4] * ygs + T[5]

        @pl.loop(0, CHUNKS_PER_ROW)
        def _chunk(cc):
            j0 = cc * K
            for g in range(GROUPS):
                xgv = grid_v[pl.ds(j0 + g * L, L)]
                xs = T[0] * xgv + xb
                ys = T[3] * xgv + yb
                xf = 0.5 * (xs + 1.0) * float(W)
                yf = 0.5 * (ys + 1.0) * float(H)
                x0 = xf.astype(jnp.int32)
                y0 = yf.astype(jnp.int32)
                x0c = jnp.clip(x0, 0, W - 1)
                x1c = jnp.clip(x0 + 1, 0, W - 1)
                y0c = jnp.clip(y0, 0, H - 1)
                y1c = jnp.clip(y0 + 1, 0, H - 1)
                zero = jnp.zeros((L,), jnp.float32)
                xdeg = x0c == x1c
                ydeg = y0c == y1c
                ax = jnp.where(xdeg, zero, x1c.astype(jnp.float32) - xf)
                cx = jnp.where(xdeg, zero, xf - x0c.astype(jnp.float32))
                ay = jnp.where(ydeg, zero, y1c.astype(jnp.float32) - yf)
                by = jnp.where(ydeg, zero, yf - y0c.astype(jnp.float32))
                sl = pl.ds(g * L, L)
                wa_v[sl] = ax * ay
                wb_v[sl] = ax * by
                wc_v[sl] = cx * ay
                wd_v[sl] = cx * by
                rowa = base + y0c * W
                rowb = base + y1c * W
                ia_v[sl] = rowa + x0c
                ib_v[sl] = rowb + x0c
                ic_v[sl] = rowa + x1c
                id_v[sl] = rowb + x1c

            cps = [pltpu.async_copy(x_hbm.at[ia_v], va_v, sem),
                   pltpu.async_copy(x_hbm.at[ib_v], vb_v, sem),
                   pltpu.async_copy(x_hbm.at[ic_v], vc_v, sem),
                   pltpu.async_copy(x_hbm.at[id_v], vd_v, sem)]
            for cp in cps:
                cp.wait()

            @pl.loop(0, GROUPS)
            def _grp(g):
                slg = pl.ds(g * L, L)
                wav = wa_v[slg]
                wbv = wb_v[slg]
                wcv = wc_v[slg]
                wdv = wd_v[slg]
                for k in range(L):
                    p = g * L + k
                    was = _splat(wav, k)
                    wbs = _splat(wbv, k)
                    wcs = _splat(wcv, k)
                    wds = _splat(wdv, k)
                    for c6 in range(C // L):
                        slc = pl.ds(c6 * L, L)
                        o = was * va_v[p, slc] + wbs * vb_v[p, slc]
                        o = o + wcs * vc_v[p, slc]
                        o = o + wds * vd_v[p, slc]
                        o_v[p, slc] = o

            pltpu.sync_copy(o_v, out_hbm.at[pl.ds(gi * W + j0, K)])


def kernel(x, transformation):
    x_flat = x.reshape(NPIX, C)
    t_flat = transformation.T.reshape(6 * L)
    grid = jnp.linspace(-1.0, 1.0, W).astype(jnp.float32)

    mesh = plsc.VectorSubcoreMesh(
        core_axis_name="c", subcore_axis_name="s",
        num_cores=NC, num_subcores=NS)
    f = pl.kernel(
        _body,
        out_type=jax.ShapeDtypeStruct((NPIX, C), jnp.float32),
        mesh=mesh,
        compiler_params=pltpu.CompilerParams(use_tc_tiling_on_sc=False),
        scratch_types=[
            pltpu.VMEM((W,), jnp.float32),
            pltpu.VMEM((6 * L,), jnp.float32),
            pltpu.VMEM((K,), jnp.int32),
            pltpu.VMEM((K,), jnp.int32),
            pltpu.VMEM((K,), jnp.int32),
            pltpu.VMEM((K,), jnp.int32),
            pltpu.VMEM((K,), jnp.float32),
            pltpu.VMEM((K,), jnp.float32),
            pltpu.VMEM((K,), jnp.float32),
            pltpu.VMEM((K,), jnp.float32),
            pltpu.VMEM((K, C), jnp.float32),
            pltpu.VMEM((K, C), jnp.float32),
            pltpu.VMEM((K, C), jnp.float32),
            pltpu.VMEM((K, C), jnp.float32),
            pltpu.VMEM((K, C), jnp.float32),
            pltpu.SemaphoreType.DMA,
        ],
    )
    out = f(x_flat, t_flat, grid)
    return out.reshape(B, H, W, C)


# speedup vs baseline: 1.4853x; 1.1295x over previous
import jax
import jax.numpy as jnp
from jax import lax
from jax.experimental import pallas as pl
from jax.experimental.pallas import tpu as pltpu
from jax.experimental.pallas import tpu_sc as plsc

H = 384
W = 384
C = 96
B = 16
NPIX = B * H * W
NC = 2
NS = 16
L = 16
NW = NC * NS
ROWS_PER_W = (B * H) // NW
K = 96
GROUPS = K // L
CHUNKS_PER_ROW = W // K
NCH = ROWS_PER_W * CHUNKS_PER_ROW
CG = C // L

_DN = lax.GatherDimensionNumbers(
    offset_dims=(), collapsed_slice_dims=(0,), start_index_map=(0,))


def _splat(vec, i):
    idx = jnp.full((L, 1), i, dtype=jnp.int32)
    return lax.gather(vec, idx, _DN, slice_sizes=(1,),
                      mode=lax.GatherScatterMode.PROMISE_IN_BOUNDS)


def _rbf(v):
    u = lax.bitcast_convert_type(v, jnp.int32)
    r = (u + jnp.int32(0x7FFF) + ((u >> 16) & 1)) & jnp.int32(-65536)
    return lax.bitcast_convert_type(r, jnp.float32)


def _body(x_hbm, t_hbm, grid_hbm, out_hbm,
          grid_v, t_v,
          ia0, ib0, ic0, id0, ia1, ib1, ic1, id1,
          wa0, wb0, wc0, wd0, wa1, wb1, wc1, wd1,
          va0, vb0, vc0, vd0, va1, vb1, vc1, vd1,
          o0, o1, sem0, sem1, os0, os1):
    cid = lax.axis_index("c")
    sid = lax.axis_index("s")
    wid = sid * NC + cid
    b = wid // (NW // B)
    base = b * (H * W)

    pltpu.sync_copy(grid_hbm, grid_v)
    pltpu.sync_copy(t_hbm, t_v)

    for gg in range(W // L):
        slg = pl.ds(gg * L, L)
        grid_v[slg] = _rbf(grid_v[slg])

    T = [_rbf(_splat(t_v[pl.ds(r * L, L)], b)) for r in range(6)]

    sets = (
        ((ia0, ib0, ic0, id0), (wa0, wb0, wc0, wd0), (va0, vb0, vc0, vd0),
         o0, sem0, os0),
        ((ia1, ib1, ic1, id1), (wa1, wb1, wc1, wd1), (va1, vb1, vc1, vd1),
         o1, sem1, os1),
    )

    def fire(m, s):
        (iab, wab, vab, _, sem, _) = s
        r = m // CHUNKS_PER_ROW
        gi = wid * ROWS_PER_W + r
        iy = gi % H
        a16 = (iy // L) * L
        ygs = _splat(grid_v[pl.ds(a16, L)], iy % L)
        xb = T[1] * ygs + T[2]
        yb = T[4] * ygs + T[5]
        j0 = (m % CHUNKS_PER_ROW) * K
        for g in range(GROUPS):
            xgv = grid_v[pl.ds(j0 + g * L, L)]
            xs = T[0] * xgv + xb
            ys = T[3] * xgv + yb
            xf = 0.5 * (xs + 1.0) * float(W)
            yf = 0.5 * (ys + 1.0) * float(H)
            x0 = xf.astype(jnp.int32)
            y0 = yf.astype(jnp.int32)
            x0c = jnp.clip(x0, 0, W - 1)
            x1c = jnp.clip(x0 + 1, 0, W - 1)
            y0c = jnp.clip(y0, 0, H - 1)
            y1c = jnp.clip(y0 + 1, 0, H - 1)
            zero = jnp.zeros((L,), jnp.float32)
            xdeg = x0c == x1c
            ydeg = y0c == y1c
            ax = jnp.where(xdeg, zero, x1c.astype(jnp.float32) - xf)
            cx = jnp.where(xdeg, zero, xf - x0c.astype(jnp.float32))
            ay = jnp.where(ydeg, zero, y1c.astype(jnp.float32) - yf)
            by = jnp.where(ydeg, zero, yf - y0c.astype(jnp.float32))
            sl = pl.ds(g * L, L)
            wab[0][sl] = ax * ay
            wab[1][sl] = ax * by
            wab[2][sl] = cx * ay
            wab[3][sl] = cx * by
            rowa = base + y0c * W
            rowb = base + y1c * W
            iab[0][sl] = rowa + x0c
            iab[1][sl] = rowb + x0c
            iab[2][sl] = rowa + x1c
            iab[3][sl] = rowb + x1c
        for q in range(4):
            pltpu.async_copy(x_hbm.at[iab[q]], vab[q], sem)

    def wait_gathers(s):
        (iab, _, vab, _, sem, _) = s
        for q in range(4):
            pltpu.make_async_copy(x_hbm.at[iab[q]], vab[q], sem).wait()

    def combine_and_store(m, s):
        (_, wab, vab, ov, _, osem) = s
        va, vb, vc, vd = vab

        @pl.loop(0, GROUPS)
        def _grp(g):
            slg = pl.ds(g * L, L)
            wav = wab[0][slg]
            wbv = wab[1][slg]
            wcv = wab[2][slg]
            wdv = wab[3][slg]
            for k in range(L):
                p = g * L + k
                was = _splat(wav, k)
                wbs = _splat(wbv, k)
                wcs = _splat(wcv, k)
                wds = _splat(wdv, k)
                for c6 in range(CG):
                    slc = pl.ds(c6 * L, L)
                    o = was * va[p, slc] + wbs * vb[p, slc]
                    o = o + wcs * vc[p, slc]
                    o = o + wds * vd[p, slc]
                    ov[p, slc] = o

        r = m // CHUNKS_PER_ROW
        gi = wid * ROWS_PER_W + r
        pix = gi * W + (m % CHUNKS_PER_ROW) * K
        pltpu.async_copy(ov, out_hbm.at[pl.ds(pix, K)], osem)

    def wait_out(s):
        (_, _, _, ov, _, osem) = s
        r = 0
        pltpu.make_async_copy(ov, out_hbm.at[pl.ds(r, K)], osem).wait()

    fire(jnp.int32(0), sets[0])

    @pl.loop(0, NCH // 2)
    def _pipe(mm):
        ca = 2 * mm
        cb = 2 * mm + 1
        fire(cb, sets[1])
        wait_gathers(sets[0])
        combine_and_store(ca, sets[0])
        cn = jnp.minimum(ca + 2, NCH - 1)
        fire(cn, sets[0])
        wait_gathers(sets[1])
        combine_and_store(cb, sets[1])
        wait_out(sets[0])
        wait_out(sets[1])

    wait_gathers(sets[0])


def kernel(x, transformation):
    x_flat = x.reshape(NPIX, C)
    t_flat = transformation.T.reshape(6 * L)
    grid = jnp.linspace(-1.0, 1.0, W).astype(jnp.float32)

    mesh = plsc.VectorSubcoreMesh(
        core_axis_name="c", subcore_axis_name="s",
        num_cores=NC, num_subcores=NS)
    ivm = lambda: pltpu.VMEM((K,), jnp.int32)
    fvm = lambda: pltpu.VMEM((K,), jnp.float32)
    rvm = lambda: pltpu.VMEM((K, C), jnp.float32)
    f = pl.kernel(
        _body,
        out_type=jax.ShapeDtypeStruct((NPIX, C), jnp.float32),
        mesh=mesh,
        compiler_params=pltpu.CompilerParams(use_tc_tiling_on_sc=False),
        scratch_types=[
            pltpu.VMEM((W,), jnp.float32),
            pltpu.VMEM((6 * L,), jnp.float32),
            ivm(), ivm(), ivm(), ivm(),
            ivm(), ivm(), ivm(), ivm(),
            fvm(), fvm(), fvm(), fvm(),
            fvm(), fvm(), fvm(), fvm(),
            rvm(), rvm(), rvm(), rvm(),
            rvm(), rvm(), rvm(), rvm(),
            rvm(), rvm(),
            pltpu.SemaphoreType.DMA,
            pltpu.SemaphoreType.DMA,
            pltpu.SemaphoreType.DMA,
            pltpu.SemaphoreType.DMA,
        ],
    )
    out = f(x_flat, t_flat, grid)
    return out.reshape(B, H, W, C)
